# Initial kernel scaffold; baseline (speedup 1.0000x reference)
import jax
import jax.numpy as jnp
from jax.experimental import pallas as pl


def kernel(x, codebook_w, enc_W, enc_b, dec_W, dec_b):
    raise NotImplementedError("write your pallas kernel here")


# speedup vs baseline: 16.4712x; 16.4712x over previous
import functools

import jax
import jax.numpy as jnp
from jax.experimental import pallas as pl
from jax.experimental.pallas import tpu as pltpu

CB = 8192
H = 1024
OBJ = 32
B = 64
M = B * OBJ
G = CB // 128

_BIG = 1 << 30


def _enc_body(x_ref, w_ref, b_ref, o_ref):
    p = jax.lax.dot_general(x_ref[...], w_ref[...], (((1,), (1,)), ((), ())),
                            preferred_element_type=jnp.float32)
    o_ref[...] = p + b_ref[...][None, :]


_NT_E = 8
_BLK_E = (OBJ * H) // _NT_E

_enc = pl.pallas_call(
    _enc_body,
    grid=(_NT_E,),
    in_specs=[
        pl.BlockSpec((B, H), lambda i: (0, 0)),
        pl.BlockSpec((_BLK_E, H), lambda i: (i, 0)),
        pl.BlockSpec((_BLK_E,), lambda i: (i,)),
    ],
    out_specs=pl.BlockSpec((B, _BLK_E), lambda i: (0, i)),
    out_shape=jax.ShapeDtypeStruct((B, OBJ * H), jnp.float32),
)


def _prod_body(ef_ref, cw_ref, o_ref):
    o_ref[...] = jax.lax.dot_general(
        ef_ref[...], cw_ref[...], (((1,), (1,)), ((), ())),
        preferred_element_type=jnp.float32)


_MT, _NT = 256, 2048

_prod = pl.pallas_call(
    _prod_body,
    grid=(M // _MT, CB // _NT),
    in_specs=[
        pl.BlockSpec((_MT, H), lambda i, j: (i, 0)),
        pl.BlockSpec((_NT, H), lambda i, j: (j, 0)),
    ],
    out_specs=pl.BlockSpec((_MT, _NT), lambda i, j: (i, j)),
    out_shape=jax.ShapeDtypeStruct((M, CB), jnp.float32),
)


def _lsa_body(cost_ref, out_ref, u_s, c4r_s, sr_s):
    fiota = (jax.lax.broadcasted_iota(jnp.int32, (G, 128), 0) * 128
             + jax.lax.broadcasted_iota(jnp.int32, (G, 128), 1))
    inf = jnp.float32(jnp.inf)

    def extract_f(vec, pos):
        return jnp.min(jnp.where(fiota == pos, vec, inf))

    def extract_i(vec, pos):
        return jnp.sum(jnp.where(fiota == pos, vec, 0))

    def init_smem(rr, carry):
        u_s[rr] = jnp.float32(0.0)
        c4r_s[rr] = jnp.int32(-1)
        return carry

    jax.lax.fori_loop(0, OBJ, init_smem, 0)

    def outer(cur_row, state):
        v, row4col = state
        shortest0 = jnp.full((G, 128), inf, jnp.float32)
        path0 = jnp.full((G, 128), -1, jnp.int32)
        remaining0 = jnp.ones((G, 128), jnp.int32)

        def clear_sr(rr, carry):
            sr_s[rr] = jnp.int32(0)
            return carry

        jax.lax.fori_loop(0, OBJ, clear_sr, 0)

        def search_cond(s):
            return s[1] == -1

        def search_body(s):
            i, sink, min_val, nsr, shortest, path, remaining = s
            sr_s[i] = jnp.int32(1)
            crow = cost_ref[0, i]
            u_i = u_s[i]
            r = ((min_val + crow) - u_i) - v
            rem_b = remaining != 0
            upd = rem_b & (r < shortest)
            shortest = jnp.where(upd, r, shortest)
            path = jnp.where(upd, i, path)
            masked = jnp.where(rem_b, shortest, inf)
            mv = jnp.min(masked)
            j = jnp.min(jnp.where(masked == mv, fiota, _BIG))
            remaining = jnp.where(fiota == j, 0, remaining)
            r4c_j = extract_i(row4col, j)
            hit = r4c_j == -1
            sink = jnp.where(hit, j, sink)
            i = jnp.where(hit, i, r4c_j)
            return (i, sink, mv, nsr + 1, shortest, path, remaining)

        init = (jnp.int32(0) + cur_row, jnp.int32(-1), jnp.float32(0.0),
                jnp.int32(0), shortest0, path0, remaining0)
        _, sink, min_val, nsr, shortest, path, remaining = jax.lax.while_loop(
            search_cond, search_body, init)

        u_s[cur_row] = u_s[cur_row] + min_val

        @pl.when(nsr > 1)
        def _():
            def body(rr, carry):
                take = (sr_s[rr] > 0) & (rr != cur_row)

                @pl.when(take)
                def _():
                    s_val = extract_f(shortest, c4r_s[rr])
                    u_s[rr] = u_s[rr] + (min_val - s_val)

                return carry

            jax.lax.fori_loop(0, OBJ, body, 0)

        v = jnp.where(remaining == 0, v - (min_val - shortest), v)

        def aug_cond(s):
            return ~s[2]

        def aug_body(s):
            j, row4col, done = s
            i = extract_i(path, j)
            row4col = jnp.where(fiota == j, i, row4col)
            nj = c4r_s[i]
            c4r_s[i] = j
            done = i == cur_row
            return (nj, row4col, done)

        _, row4col, _ = jax.lax.while_loop(
            aug_cond, aug_body, (sink, row4col, jnp.bool_(False)))
        return (v, row4col)

    v0 = jnp.zeros((G, 128), jnp.float32)
    row4col0 = jnp.full((G, 128), -1, jnp.int32)
    jax.lax.fori_loop(0, OBJ, outer, (v0, row4col0))

    def writeout(rr, carry):
        out_ref[0, 0, rr] = c4r_s[rr]
        return carry

    jax.lax.fori_loop(0, OBJ, writeout, 0)


_lsa_call = pl.pallas_call(
    _lsa_body,
    grid=(B,),
    in_specs=[pl.BlockSpec((1, OBJ, G, 128), lambda s: (s, 0, 0, 0))],
    out_specs=pl.BlockSpec((1, 1, OBJ), lambda s: (s, 0, 0),
                           memory_space=pltpu.SMEM),
    out_shape=jax.ShapeDtypeStruct((B, 1, OBJ), jnp.int32),
    scratch_shapes=[
        pltpu.SMEM((OBJ,), jnp.float32),
        pltpu.SMEM((OBJ,), jnp.int32),
        pltpu.SMEM((OBJ,), jnp.int32),
    ],
)


def _gather_body(idx_ref, cw_ref, o_ref):
    s = pl.program_id(0)

    def body(r, _):
        o_ref[0, r] = cw_ref[idx_ref[s * OBJ + r]]
        return 0

    jax.lax.fori_loop(0, OBJ, body, 0)


_gather = pl.pallas_call(
    _gather_body,
    grid_spec=pltpu.PrefetchScalarGridSpec(
        num_scalar_prefetch=1,
        grid=(B,),
        in_specs=[pl.BlockSpec((CB, 8, 128), lambda s, idx: (0, 0, 0))],
        out_specs=pl.BlockSpec((1, OBJ, 8, 128), lambda s, idx: (s, 0, 0, 0)),
    ),
    out_shape=jax.ShapeDtypeStruct((B, OBJ, 8, 128), jnp.float32),
)


def _dec_body(q_ref, w_ref, b_ref, o_ref):
    k = pl.program_id(0)
    nk = pl.num_programs(0)
    p = jax.lax.dot_general(q_ref[...], w_ref[...], (((1,), (1,)), ((), ())),
                            preferred_element_type=jnp.float32)

    @pl.when(k == 0)
    def _():
        o_ref[...] = p

    @pl.when(k > 0)
    def _():
        o_ref[...] += p

    @pl.when(k == nk - 1)
    def _():
        o_ref[...] += b_ref[...][None, :]


_NK_D = 8
_BLK_D = (OBJ * H) // _NK_D

_dec = pl.pallas_call(
    _dec_body,
    grid=(_NK_D,),
    in_specs=[
        pl.BlockSpec((B, _BLK_D), lambda k: (0, k)),
        pl.BlockSpec((H, _BLK_D), lambda k: (0, k)),
        pl.BlockSpec((H,), lambda k: (0,)),
    ],
    out_specs=pl.BlockSpec((B, H), lambda k: (0, 0)),
    out_shape=jax.ShapeDtypeStruct((B, H), jnp.float32),
)


def kernel(x, codebook_w, enc_W, enc_b, dec_W, dec_b):
    e2d = _enc(x, enc_W, enc_b)
    ef = e2d.reshape(M, H)
    p = _prod(ef, codebook_w)
    se = jnp.sum(ef ** 2, axis=1, keepdims=True)
    sc = jnp.sum(codebook_w ** 2, axis=1)
    dist = jnp.sqrt(se + sc - 2.0 * p)
    idx = _lsa_call(dist.reshape(B, OBJ, G, 128))
    idx_flat = idx.reshape(M)
    q = _gather(idx_flat, codebook_w.reshape(CB, 8, 128))
    q = q.reshape(B, OBJ, H)
    out = _dec(q.reshape(B, OBJ * H), dec_W, dec_b)
    e = e2d.reshape(B, OBJ, H)
    return (out, q, e)


# speedup vs baseline: 23.1948x; 1.4082x over previous
import functools

import jax
import jax.numpy as jnp
from jax.experimental import pallas as pl
from jax.experimental.pallas import tpu as pltpu

CB = 8192
H = 1024
OBJ = 32
B = 64
M = B * OBJ
G = CB // 128

_BIG = 1 << 30


def _enc_body(x_ref, w_ref, b_ref, o_ref):
    p = jax.lax.dot_general(x_ref[...], w_ref[...], (((1,), (1,)), ((), ())),
                            preferred_element_type=jnp.float32)
    o_ref[...] = p + b_ref[...][None, :]


_NT_E = 8
_BLK_E = (OBJ * H) // _NT_E

_enc = pl.pallas_call(
    _enc_body,
    grid=(_NT_E,),
    in_specs=[
        pl.BlockSpec((B, H), lambda i: (0, 0)),
        pl.BlockSpec((_BLK_E, H), lambda i: (i, 0)),
        pl.BlockSpec((_BLK_E,), lambda i: (i,)),
    ],
    out_specs=pl.BlockSpec((B, _BLK_E), lambda i: (0, i)),
    out_shape=jax.ShapeDtypeStruct((B, OBJ * H), jnp.float32),
)


def _prod_body(ef_ref, cw_ref, o_ref):
    o_ref[...] = jax.lax.dot_general(
        ef_ref[...], cw_ref[...], (((1,), (1,)), ((), ())),
        preferred_element_type=jnp.float32)


_MT, _NT = 256, 2048

_prod = pl.pallas_call(
    _prod_body,
    grid=(M // _MT, CB // _NT),
    in_specs=[
        pl.BlockSpec((_MT, H), lambda i, j: (i, 0)),
        pl.BlockSpec((_NT, H), lambda i, j: (j, 0)),
    ],
    out_specs=pl.BlockSpec((_MT, _NT), lambda i, j: (i, j)),
    out_shape=jax.ShapeDtypeStruct((M, CB), jnp.float32),
)


def _lsa_body(cost_ref, out_ref, u_s, c4r_s, sr_s):
    fiota = (jax.lax.broadcasted_iota(jnp.int32, (G, 128), 0) * 128
             + jax.lax.broadcasted_iota(jnp.int32, (G, 128), 1))
    inf = jnp.float32(jnp.inf)

    def extract_f(vec, pos):
        return jnp.min(jnp.where(fiota == pos, vec, inf))

    def extract_i(vec, pos):
        return jnp.sum(jnp.where(fiota == pos, vec, 0))

    def init_smem(rr, carry):
        u_s[rr] = jnp.float32(0.0)
        c4r_s[rr] = jnp.int32(-1)
        return carry

    jax.lax.fori_loop(0, OBJ, init_smem, 0)

    def outer(cur_row, state):
        v, row4col = state
        crow0 = cost_ref[0, cur_row]
        r0 = crow0 - v
        mv0 = jnp.min(r0)
        j0 = jnp.min(jnp.where(r0 == mv0, fiota, _BIG))
        taken = extract_i(row4col, j0) != -1

        def fast(v, row4col):
            u_s[cur_row] = u_s[cur_row] + mv0
            c4r_s[cur_row] = j0
            row4col = jnp.where(fiota == j0, cur_row, row4col)
            return (v, row4col)

        def general(v, row4col):
            shortest0 = jnp.full((G, 128), inf, jnp.float32)
            path0 = jnp.full((G, 128), -1, jnp.int32)
            remaining0 = jnp.ones((G, 128), jnp.int32)

            def clear_sr(rr, carry):
                sr_s[rr] = jnp.int32(0)
                return carry

            jax.lax.fori_loop(0, OBJ, clear_sr, 0)

            def search_cond(s):
                return s[1] == -1

            def search_body(s):
                i, sink, min_val, nsr, shortest, path, remaining = s
                sr_s[i] = jnp.int32(1)
                crow = cost_ref[0, i]
                u_i = u_s[i]
                r = ((min_val + crow) - u_i) - v
                rem_b = remaining != 0
                upd = rem_b & (r < shortest)
                shortest = jnp.where(upd, r, shortest)
                path = jnp.where(upd, i, path)
                masked = jnp.where(rem_b, shortest, inf)
                mv = jnp.min(masked)
                j = jnp.min(jnp.where(masked == mv, fiota, _BIG))
                remaining = jnp.where(fiota == j, 0, remaining)
                r4c_j = extract_i(row4col, j)
                hit = r4c_j == -1
                sink = jnp.where(hit, j, sink)
                i = jnp.where(hit, i, r4c_j)
                return (i, sink, mv, nsr + 1, shortest, path, remaining)

            init = (jnp.int32(0) + cur_row, jnp.int32(-1), jnp.float32(0.0),
                    jnp.int32(0), shortest0, path0, remaining0)
            _, sink, min_val, nsr, shortest, path, remaining = \
                jax.lax.while_loop(search_cond, search_body, init)

            u_s[cur_row] = u_s[cur_row] + min_val

            @pl.when(nsr > 1)
            def _():
                def body(rr, carry):
                    take = (sr_s[rr] > 0) & (rr != cur_row)

                    @pl.when(take)
                    def _():
                        s_val = extract_f(shortest, c4r_s[rr])
                        u_s[rr] = u_s[rr] + (min_val - s_val)

                    return carry

                jax.lax.fori_loop(0, OBJ, body, 0)

            vn = jnp.where(remaining == 0, v - (min_val - shortest), v)

            def aug_cond(s):
                return ~s[2]

            def aug_body(s):
                j, row4col, done = s
                i = extract_i(path, j)
                row4col = jnp.where(fiota == j, i, row4col)
                nj = c4r_s[i]
                c4r_s[i] = j
                done = i == cur_row
                return (nj, row4col, done)

            _, row4col_n, _ = jax.lax.while_loop(
                aug_cond, aug_body, (sink, row4col, jnp.bool_(False)))
            return (vn, row4col_n)

        return jax.lax.cond(jnp.logical_not(taken), fast, general, v, row4col)

    v0 = jnp.zeros((G, 128), jnp.float32)
    row4col0 = jnp.full((G, 128), -1, jnp.int32)
    jax.lax.fori_loop(0, OBJ, outer, (v0, row4col0))

    def writeout(rr, carry):
        out_ref[0, 0, rr] = c4r_s[rr]
        return carry

    jax.lax.fori_loop(0, OBJ, writeout, 0)


_lsa_call = pl.pallas_call(
    _lsa_body,
    grid=(B,),
    in_specs=[pl.BlockSpec((1, OBJ, G, 128), lambda s: (s, 0, 0, 0))],
    out_specs=pl.BlockSpec((1, 1, OBJ), lambda s: (s, 0, 0),
                           memory_space=pltpu.SMEM),
    out_shape=jax.ShapeDtypeStruct((B, 1, OBJ), jnp.int32),
    scratch_shapes=[
        pltpu.SMEM((OBJ,), jnp.float32),
        pltpu.SMEM((OBJ,), jnp.int32),
        pltpu.SMEM((OBJ,), jnp.int32),
    ],
)


def _gather_body(idx_ref, cw_ref, o_ref):
    s = pl.program_id(0)

    def body(r, _):
        o_ref[0, r] = cw_ref[idx_ref[s * OBJ + r]]
        return 0

    jax.lax.fori_loop(0, OBJ, body, 0)


_gather = pl.pallas_call(
    _gather_body,
    grid_spec=pltpu.PrefetchScalarGridSpec(
        num_scalar_prefetch=1,
        grid=(B,),
        in_specs=[pl.BlockSpec((CB, 8, 128), lambda s, idx: (0, 0, 0))],
        out_specs=pl.BlockSpec((1, OBJ, 8, 128), lambda s, idx: (s, 0, 0, 0)),
    ),
    out_shape=jax.ShapeDtypeStruct((B, OBJ, 8, 128), jnp.float32),
)


def _dec_body(q_ref, w_ref, b_ref, o_ref):
    k = pl.program_id(0)
    nk = pl.num_programs(0)
    p = jax.lax.dot_general(q_ref[...], w_ref[...], (((1,), (1,)), ((), ())),
                            preferred_element_type=jnp.float32)

    @pl.when(k == 0)
    def _():
        o_ref[...] = p

    @pl.when(k > 0)
    def _():
        o_ref[...] += p

    @pl.when(k == nk - 1)
    def _():
        o_ref[...] += b_ref[...][None, :]


_NK_D = 8
_BLK_D = (OBJ * H) // _NK_D

_dec = pl.pallas_call(
    _dec_body,
    grid=(_NK_D,),
    in_specs=[
        pl.BlockSpec((B, _BLK_D), lambda k: (0, k)),
        pl.BlockSpec((H, _BLK_D), lambda k: (0, k)),
        pl.BlockSpec((H,), lambda k: (0,)),
    ],
    out_specs=pl.BlockSpec((B, H), lambda k: (0, 0)),
    out_shape=jax.ShapeDtypeStruct((B, H), jnp.float32),
)


def kernel(x, codebook_w, enc_W, enc_b, dec_W, dec_b):
    e2d = _enc(x, enc_W, enc_b)
    ef = e2d.reshape(M, H)
    p = _prod(ef, codebook_w)
    se = jnp.sum(ef ** 2, axis=1, keepdims=True)
    sc = jnp.sum(codebook_w ** 2, axis=1)
    dist = jnp.sqrt(se + sc - 2.0 * p)
    idx = _lsa_call(dist.reshape(B, OBJ, G, 128))
    idx_flat = idx.reshape(M)
    q = _gather(idx_flat, codebook_w.reshape(CB, 8, 128))
    q = q.reshape(B, OBJ, H)
    out = _dec(q.reshape(B, OBJ * H), dec_W, dec_b)
    e = e2d.reshape(B, OBJ, H)
    return (out, q, e)


# speedup vs baseline: 48.6109x; 2.0958x over previous
import functools

import jax
import jax.numpy as jnp
from jax.experimental import pallas as pl
from jax.experimental.pallas import tpu as pltpu

CB = 8192
H = 1024
OBJ = 32
B = 64
M = B * OBJ
G = CB // 128

_BIG = 1 << 30


def _enc_body(x_ref, w_ref, b_ref, o_ref):
    p = jax.lax.dot_general(x_ref[...], w_ref[...], (((1,), (1,)), ((), ())),
                            preferred_element_type=jnp.float32)
    o_ref[...] = p + b_ref[...][None, :]


_NT_E = 8
_BLK_E = (OBJ * H) // _NT_E

_enc = pl.pallas_call(
    _enc_body,
    grid=(_NT_E,),
    in_specs=[
        pl.BlockSpec((B, H), lambda i: (0, 0)),
        pl.BlockSpec((_BLK_E, H), lambda i: (i, 0)),
        pl.BlockSpec((_BLK_E,), lambda i: (i,)),
    ],
    out_specs=pl.BlockSpec((B, _BLK_E), lambda i: (0, i)),
    out_shape=jax.ShapeDtypeStruct((B, OBJ * H), jnp.float32),
)


def _prod_body(ef_ref, cw_ref, o_ref):
    o_ref[...] = jax.lax.dot_general(
        ef_ref[...], cw_ref[...], (((1,), (1,)), ((), ())),
        preferred_element_type=jnp.float32)


_MT, _NT = 256, 2048

_prod = pl.pallas_call(
    _prod_body,
    grid=(M // _MT, CB // _NT),
    in_specs=[
        pl.BlockSpec((_MT, H), lambda i, j: (i, 0)),
        pl.BlockSpec((_NT, H), lambda i, j: (j, 0)),
    ],
    out_specs=pl.BlockSpec((_MT, _NT), lambda i, j: (i, j)),
    out_shape=jax.ShapeDtypeStruct((M, CB), jnp.float32),
)


from jax import lax
from jax.experimental.pallas import tpu_sc as plsc

_NC = 2
_NS = 16
_NW = _NC * _NS
_SPW = B // _NW
_NCHUNK = CB // 16


def _sc_lsa_kernel(dist_hbm, out_hbm, crow_ref, v_ref, sh_ref, path_ref,
                   rem_ref, r4c_ref, c4r_ref, u_s, sr_s, c4r_s):
    wid = lax.axis_index("s") * _NC + lax.axis_index("c")
    lane = jax.lax.broadcasted_iota(jnp.int32, (16,), 0)
    inf = jnp.float32(jnp.inf)
    inf16 = jnp.full((16,), jnp.inf, jnp.float32)
    zero16i = jnp.zeros((16,), jnp.int32)

    def ds16(c):
        return pl.ds(c * 16, 16)

    def wr16(ref, pos, val):
        sl = pl.ds((pos // 16) * 16, 16)
        ref[sl] = jnp.where(lane == (pos % 16), val, ref[sl])

    def rd16i(ref, pos):
        vec = ref[pl.ds((pos // 16) * 16, 16)]
        return jnp.sum(jnp.where(lane == (pos % 16), vec, 0))

    def rd16f(ref, pos):
        vec = ref[pl.ds((pos // 16) * 16, 16)]
        return jnp.min(jnp.where(lane == (pos % 16), vec, inf))

    for so in range(_SPW):
        sample = wid * _SPW + so

        def init_chunk(c, carry):
            v_ref[ds16(c)] = jnp.zeros((16,), jnp.float32)
            r4c_ref[ds16(c)] = jnp.full((16,), -1, jnp.int32)
            return carry

        lax.fori_loop(0, _NCHUNK, init_chunk, 0)

        def init_small(rr, carry):
            u_s[rr] = jnp.float32(0.0)
            c4r_s[rr] = jnp.int32(-1)
            return carry

        lax.fori_loop(0, OBJ, init_small, 0)
        c4r_ref[pl.ds(0, 16)] = jnp.full((16,), -1, jnp.int32)
        c4r_ref[pl.ds(16, 16)] = jnp.full((16,), -1, jnp.int32)

        def outer(cur_row, carry):
            pltpu.sync_copy(dist_hbm.at[sample, cur_row], crow_ref)

            def fp_chunk(c, st):
                rm, rc = st
                x = crow_ref[ds16(c)] - v_ref[ds16(c)]
                lt = x < rm
                rm = jnp.where(lt, x, rm)
                rc = jnp.where(lt, c, rc)
                return (rm, rc)

            rm, rc = lax.fori_loop(0, _NCHUNK, fp_chunk, (inf16, zero16i))
            mv0 = jnp.min(rm)
            j0 = jnp.min(jnp.where(rm == mv0, rc * 16 + lane, _BIG))
            taken = rd16i(r4c_ref, j0) != -1

            @pl.when(jnp.logical_not(taken))
            def _():
                u_s[cur_row] = u_s[cur_row] + mv0
                c4r_s[cur_row] = j0
                wr16(c4r_ref, cur_row, j0)
                wr16(r4c_ref, j0, cur_row)

            @pl.when(taken)
            def _():
                def ginit(c, carryg):
                    sh_ref[ds16(c)] = inf16
                    path_ref[ds16(c)] = jnp.full((16,), -1, jnp.int32)
                    rem_ref[ds16(c)] = jnp.full((16,), 1, jnp.int32)
                    return carryg

                lax.fori_loop(0, _NCHUNK, ginit, 0)

                def clear_sr(rr, carryg):
                    sr_s[rr] = jnp.int32(0)
                    return carryg

                lax.fori_loop(0, OBJ, clear_sr, 0)

                def search_cond(s):
                    return s[1] == -1

                def search_body(s):
                    i, sink, min_val, nsr = s
                    sr_s[i] = jnp.int32(1)
                    pltpu.sync_copy(dist_hbm.at[sample, i], crow_ref)
                    u_i = u_s[i]

                    def schunk(c, st):
                        srm, src_ = st
                        x = ((min_val + crow_ref[ds16(c)]) - u_i) - v_ref[ds16(c)]
                        sh = sh_ref[ds16(c)]
                        remc = rem_ref[ds16(c)]
                        remb = remc != 0
                        upd = remb & (x < sh)
                        sh2 = jnp.where(upd, x, sh)
                        sh_ref[ds16(c)] = sh2
                        path_ref[ds16(c)] = jnp.where(upd, i, path_ref[ds16(c)])
                        masked = jnp.where(remb, sh2, inf)
                        lt = masked < srm
                        srm = jnp.where(lt, masked, srm)
                        src_ = jnp.where(lt, c, src_)
                        return (srm, src_)

                    srm, src_ = lax.fori_loop(0, _NCHUNK, schunk,
                                              (inf16, zero16i))
                    mv = jnp.min(srm)
                    j = jnp.min(jnp.where(srm == mv, src_ * 16 + lane, _BIG))
                    wr16(rem_ref, j, jnp.int32(0))
                    r4c_j = rd16i(r4c_ref, j)
                    hit = r4c_j == -1
                    sink = jnp.where(hit, j, sink)
                    i = jnp.where(hit, i, r4c_j)
                    return (i, sink, mv, nsr + 1)

                init = (jnp.int32(0) + cur_row, jnp.int32(-1),
                        jnp.float32(0.0), jnp.int32(0))
                _, sink, min_val, nsr = lax.while_loop(
                    search_cond, search_body, init)

                u_s[cur_row] = u_s[cur_row] + min_val

                @pl.when(nsr > 1)
                def _():
                    def ex_body(rr, carryg):
                        take = (sr_s[rr] > 0) & (rr != cur_row)

                        @pl.when(take)
                        def _():
                            s_val = rd16f(sh_ref, c4r_s[rr])
                            u_s[rr] = u_s[rr] + (min_val - s_val)

                        return carryg

                    lax.fori_loop(0, OBJ, ex_body, 0)

                def vchunk(c, carryg):
                    t = min_val - sh_ref[ds16(c)]
                    remc = rem_ref[ds16(c)]
                    v_ref[ds16(c)] = jnp.where(remc == 0,
                                               v_ref[ds16(c)] - t,
                                               v_ref[ds16(c)])
                    return carryg

                lax.fori_loop(0, _NCHUNK, vchunk, 0)

                def aug_cond(s):
                    return jnp.logical_not(s[1])

                def aug_body(s):
                    j, done = s
                    i = rd16i(path_ref, j)
                    wr16(r4c_ref, j, i)
                    nj = c4r_s[i]
                    c4r_s[i] = j
                    wr16(c4r_ref, i, j)
                    done = i == cur_row
                    return (nj, done)

                lax.while_loop(aug_cond, aug_body, (sink, jnp.bool_(False)))

            return carry

        lax.fori_loop(0, OBJ, outer, 0)
        pltpu.sync_copy(c4r_ref, out_hbm.at[sample])


_lsa_call_sc = functools.partial(
    pl.kernel,
    mesh=plsc.VectorSubcoreMesh(core_axis_name="c", subcore_axis_name="s"),
    compiler_params=pltpu.CompilerParams(needs_layout_passes=False),
    out_type=jax.ShapeDtypeStruct((B, OBJ), jnp.int32),
    scratch_types=[
        pltpu.VMEM((CB,), jnp.float32),
        pltpu.VMEM((CB,), jnp.float32),
        pltpu.VMEM((CB,), jnp.float32),
        pltpu.VMEM((CB,), jnp.int32),
        pltpu.VMEM((CB,), jnp.int32),
        pltpu.VMEM((CB,), jnp.int32),
        pltpu.VMEM((OBJ,), jnp.int32),
        pltpu.SMEM((OBJ,), jnp.float32),
        pltpu.SMEM((OBJ,), jnp.int32),
        pltpu.SMEM((OBJ,), jnp.int32),
    ],
)(_sc_lsa_kernel)


def _gather_body(idx_ref, cw_ref, o_ref):
    s = pl.program_id(0)

    def body(r, _):
        o_ref[0, r] = cw_ref[idx_ref[s * OBJ + r]]
        return 0

    jax.lax.fori_loop(0, OBJ, body, 0)


_gather = pl.pallas_call(
    _gather_body,
    grid_spec=pltpu.PrefetchScalarGridSpec(
        num_scalar_prefetch=1,
        grid=(B,),
        in_specs=[pl.BlockSpec((CB, 8, 128), lambda s, idx: (0, 0, 0))],
        out_specs=pl.BlockSpec((1, OBJ, 8, 128), lambda s, idx: (s, 0, 0, 0)),
    ),
    out_shape=jax.ShapeDtypeStruct((B, OBJ, 8, 128), jnp.float32),
)


def _dec_body(q_ref, w_ref, b_ref, o_ref):
    k = pl.program_id(0)
    nk = pl.num_programs(0)
    p = jax.lax.dot_general(q_ref[...], w_ref[...], (((1,), (1,)), ((), ())),
                            preferred_element_type=jnp.float32)

    @pl.when(k == 0)
    def _():
        o_ref[...] = p

    @pl.when(k > 0)
    def _():
        o_ref[...] += p

    @pl.when(k == nk - 1)
    def _():
        o_ref[...] += b_ref[...][None, :]


_NK_D = 8
_BLK_D = (OBJ * H) // _NK_D

_dec = pl.pallas_call(
    _dec_body,
    grid=(_NK_D,),
    in_specs=[
        pl.BlockSpec((B, _BLK_D), lambda k: (0, k)),
        pl.BlockSpec((H, _BLK_D), lambda k: (0, k)),
        pl.BlockSpec((H,), lambda k: (0,)),
    ],
    out_specs=pl.BlockSpec((B, H), lambda k: (0, 0)),
    out_shape=jax.ShapeDtypeStruct((B, H), jnp.float32),
)


def kernel(x, codebook_w, enc_W, enc_b, dec_W, dec_b):
    e2d = _enc(x, enc_W, enc_b)
    ef = e2d.reshape(M, H)
    p = _prod(ef, codebook_w)
    se = jnp.sum(ef ** 2, axis=1, keepdims=True)
    sc = jnp.sum(codebook_w ** 2, axis=1)
    dist = jnp.sqrt(se + sc - 2.0 * p)
    idx = _lsa_call_sc(dist.reshape(B, OBJ, CB))
    idx_flat = idx.reshape(M)
    q = _gather(idx_flat, codebook_w.reshape(CB, 8, 128))
    q = q.reshape(B, OBJ, H)
    out = _dec(q.reshape(B, OBJ * H), dec_W, dec_b)
    e = e2d.reshape(B, OBJ, H)
    return (out, q, e)


# speedup vs baseline: 57.1152x; 1.1749x over previous
import functools

import jax
import jax.numpy as jnp
from jax.experimental import pallas as pl
from jax.experimental.pallas import tpu as pltpu

CB = 8192
H = 1024
OBJ = 32
B = 64
M = B * OBJ
G = CB // 128

_BIG = 1 << 30


def _enc_body(x_ref, w_ref, b_ref, o_ref):
    p = jax.lax.dot_general(x_ref[...], w_ref[...], (((1,), (1,)), ((), ())),
                            preferred_element_type=jnp.float32)
    o_ref[...] = p + b_ref[...][None, :]


_NT_E = 8
_BLK_E = (OBJ * H) // _NT_E

_enc = pl.pallas_call(
    _enc_body,
    grid=(_NT_E,),
    in_specs=[
        pl.BlockSpec((B, H), lambda i: (0, 0)),
        pl.BlockSpec((_BLK_E, H), lambda i: (i, 0)),
        pl.BlockSpec((_BLK_E,), lambda i: (i,)),
    ],
    out_specs=pl.BlockSpec((B, _BLK_E), lambda i: (0, i)),
    out_shape=jax.ShapeDtypeStruct((B, OBJ * H), jnp.float32),
)


def _prod_body(ef_ref, cw_ref, o_ref):
    o_ref[...] = jax.lax.dot_general(
        ef_ref[...], cw_ref[...], (((1,), (1,)), ((), ())),
        preferred_element_type=jnp.float32)


_MT, _NT = 256, 2048

_prod = pl.pallas_call(
    _prod_body,
    grid=(M // _MT, CB // _NT),
    in_specs=[
        pl.BlockSpec((_MT, H), lambda i, j: (i, 0)),
        pl.BlockSpec((_NT, H), lambda i, j: (j, 0)),
    ],
    out_specs=pl.BlockSpec((_MT, _NT), lambda i, j: (i, j)),
    out_shape=jax.ShapeDtypeStruct((M, CB), jnp.float32),
)


from jax import lax
from jax.experimental.pallas import tpu_sc as plsc

_NC = 2
_NS = 16
_NW = _NC * _NS
_SPW = B // _NW
_NCHUNK = CB // 16


def _sc_lsa_kernel(dist_hbm, out_hbm, crow_ref, v_ref, sh_ref, path_ref,
                   rem_ref, r4c_ref, c4r_ref, u_s, sr_s, c4r_s):
    wid = lax.axis_index("s") * _NC + lax.axis_index("c")
    lane = jax.lax.broadcasted_iota(jnp.int32, (16,), 0)
    inf = jnp.float32(jnp.inf)
    inf16 = jnp.full((16,), jnp.inf, jnp.float32)
    zero16i = jnp.zeros((16,), jnp.int32)

    def ds16(c):
        return pl.ds(c * 16, 16)

    def wr16(ref, pos, val):
        sl = pl.ds((pos // 16) * 16, 16)
        ref[sl] = jnp.where(lane == (pos % 16), val, ref[sl])

    def rd16i(ref, pos):
        vec = ref[pl.ds((pos // 16) * 16, 16)]
        return jnp.sum(jnp.where(lane == (pos % 16), vec, 0))

    def rd16f(ref, pos):
        vec = ref[pl.ds((pos // 16) * 16, 16)]
        return jnp.min(jnp.where(lane == (pos % 16), vec, inf))

    for so in range(_SPW):
        sample = wid * _SPW + so

        def init_chunk(c, carry):
            v_ref[ds16(c)] = jnp.zeros((16,), jnp.float32)
            r4c_ref[ds16(c)] = jnp.full((16,), -1, jnp.int32)
            return carry

        lax.fori_loop(0, _NCHUNK, init_chunk, 0, unroll=8)

        def init_small(rr, carry):
            u_s[rr] = jnp.float32(0.0)
            c4r_s[rr] = jnp.int32(-1)
            return carry

        lax.fori_loop(0, OBJ, init_small, 0)
        c4r_ref[pl.ds(0, 16)] = jnp.full((16,), -1, jnp.int32)
        c4r_ref[pl.ds(16, 16)] = jnp.full((16,), -1, jnp.int32)

        def outer(cur_row, carry):
            pltpu.sync_copy(dist_hbm.at[sample, cur_row], crow_ref)

            def fp_chunk(c, st):
                rm, rc = st
                x = crow_ref[ds16(c)] - v_ref[ds16(c)]
                lt = x < rm
                rm = jnp.where(lt, x, rm)
                rc = jnp.where(lt, c, rc)
                return (rm, rc)

            rm, rc = lax.fori_loop(0, _NCHUNK, fp_chunk, (inf16, zero16i),
                                   unroll=8)
            mv0 = jnp.min(rm)
            j0 = jnp.min(jnp.where(rm == mv0, rc * 16 + lane, _BIG))
            taken = rd16i(r4c_ref, j0) != -1

            @pl.when(jnp.logical_not(taken))
            def _():
                u_s[cur_row] = u_s[cur_row] + mv0
                c4r_s[cur_row] = j0
                wr16(c4r_ref, cur_row, j0)
                wr16(r4c_ref, j0, cur_row)

            @pl.when(taken)
            def _():
                def ginit(c, carryg):
                    sh_ref[ds16(c)] = inf16
                    path_ref[ds16(c)] = jnp.full((16,), -1, jnp.int32)
                    rem_ref[ds16(c)] = jnp.full((16,), 1, jnp.int32)
                    return carryg

                lax.fori_loop(0, _NCHUNK, ginit, 0, unroll=8)

                def clear_sr(rr, carryg):
                    sr_s[rr] = jnp.int32(0)
                    return carryg

                lax.fori_loop(0, OBJ, clear_sr, 0)

                def search_cond(s):
                    return s[1] == -1

                def search_body(s):
                    i, sink, min_val, nsr = s
                    sr_s[i] = jnp.int32(1)
                    pltpu.sync_copy(dist_hbm.at[sample, i], crow_ref)
                    u_i = u_s[i]

                    def schunk(c, st):
                        srm, src_ = st
                        x = ((min_val + crow_ref[ds16(c)]) - u_i) - v_ref[ds16(c)]
                        sh = sh_ref[ds16(c)]
                        remc = rem_ref[ds16(c)]
                        remb = remc != 0
                        upd = remb & (x < sh)
                        sh2 = jnp.where(upd, x, sh)
                        sh_ref[ds16(c)] = sh2
                        path_ref[ds16(c)] = jnp.where(upd, i, path_ref[ds16(c)])
                        masked = jnp.where(remb, sh2, inf)
                        lt = masked < srm
                        srm = jnp.where(lt, masked, srm)
                        src_ = jnp.where(lt, c, src_)
                        return (srm, src_)

                    srm, src_ = lax.fori_loop(0, _NCHUNK, schunk,
                                              (inf16, zero16i), unroll=4)
                    mv = jnp.min(srm)
                    j = jnp.min(jnp.where(srm == mv, src_ * 16 + lane, _BIG))
                    wr16(rem_ref, j, jnp.int32(0))
                    r4c_j = rd16i(r4c_ref, j)
                    hit = r4c_j == -1
                    sink = jnp.where(hit, j, sink)
                    i = jnp.where(hit, i, r4c_j)
                    return (i, sink, mv, nsr + 1)

                init = (jnp.int32(0) + cur_row, jnp.int32(-1),
                        jnp.float32(0.0), jnp.int32(0))
                _, sink, min_val, nsr = lax.while_loop(
                    search_cond, search_body, init)

                u_s[cur_row] = u_s[cur_row] + min_val

                @pl.when(nsr > 1)
                def _():
                    def ex_body(rr, carryg):
                        take = (sr_s[rr] > 0) & (rr != cur_row)

                        @pl.when(take)
                        def _():
                            s_val = rd16f(sh_ref, c4r_s[rr])
                            u_s[rr] = u_s[rr] + (min_val - s_val)

                        return carryg

                    lax.fori_loop(0, OBJ, ex_body, 0)

                def vchunk(c, carryg):
                    t = min_val - sh_ref[ds16(c)]
                    remc = rem_ref[ds16(c)]
                    v_ref[ds16(c)] = jnp.where(remc == 0,
                                               v_ref[ds16(c)] - t,
                                               v_ref[ds16(c)])
                    return carryg

                lax.fori_loop(0, _NCHUNK, vchunk, 0, unroll=4)

                def aug_cond(s):
                    return jnp.logical_not(s[1])

                def aug_body(s):
                    j, done = s
                    i = rd16i(path_ref, j)
                    wr16(r4c_ref, j, i)
                    nj = c4r_s[i]
                    c4r_s[i] = j
                    wr16(c4r_ref, i, j)
                    done = i == cur_row
                    return (nj, done)

                lax.while_loop(aug_cond, aug_body, (sink, jnp.bool_(False)))

            return carry

        lax.fori_loop(0, OBJ, outer, 0)
        pltpu.sync_copy(c4r_ref, out_hbm.at[sample])


_lsa_call_sc = functools.partial(
    pl.kernel,
    mesh=plsc.VectorSubcoreMesh(core_axis_name="c", subcore_axis_name="s"),
    compiler_params=pltpu.CompilerParams(needs_layout_passes=False),
    out_type=jax.ShapeDtypeStruct((B, OBJ), jnp.int32),
    scratch_types=[
        pltpu.VMEM((CB,), jnp.float32),
        pltpu.VMEM((CB,), jnp.float32),
        pltpu.VMEM((CB,), jnp.float32),
        pltpu.VMEM((CB,), jnp.int32),
        pltpu.VMEM((CB,), jnp.int32),
        pltpu.VMEM((CB,), jnp.int32),
        pltpu.VMEM((OBJ,), jnp.int32),
        pltpu.SMEM((OBJ,), jnp.float32),
        pltpu.SMEM((OBJ,), jnp.int32),
        pltpu.SMEM((OBJ,), jnp.int32),
    ],
)(_sc_lsa_kernel)


def _gather_body(idx_ref, cw_ref, o_ref):
    s = pl.program_id(0)

    def body(r, _):
        o_ref[0, r] = cw_ref[idx_ref[s * OBJ + r]]
        return 0

    jax.lax.fori_loop(0, OBJ, body, 0)


_gather = pl.pallas_call(
    _gather_body,
    grid_spec=pltpu.PrefetchScalarGridSpec(
        num_scalar_prefetch=1,
        grid=(B,),
        in_specs=[pl.BlockSpec((CB, 8, 128), lambda s, idx: (0, 0, 0))],
        out_specs=pl.BlockSpec((1, OBJ, 8, 128), lambda s, idx: (s, 0, 0, 0)),
    ),
    out_shape=jax.ShapeDtypeStruct((B, OBJ, 8, 128), jnp.float32),
)


def _dec_body(q_ref, w_ref, b_ref, o_ref):
    k = pl.program_id(0)
    nk = pl.num_programs(0)
    p = jax.lax.dot_general(q_ref[...], w_ref[...], (((1,), (1,)), ((), ())),
                            preferred_element_type=jnp.float32)

    @pl.when(k == 0)
    def _():
        o_ref[...] = p

    @pl.when(k > 0)
    def _():
        o_ref[...] += p

    @pl.when(k == nk - 1)
    def _():
        o_ref[...] += b_ref[...][None, :]


_NK_D = 8
_BLK_D = (OBJ * H) // _NK_D

_dec = pl.pallas_call(
    _dec_body,
    grid=(_NK_D,),
    in_specs=[
        pl.BlockSpec((B, _BLK_D), lambda k: (0, k)),
        pl.BlockSpec((H, _BLK_D), lambda k: (0, k)),
        pl.BlockSpec((H,), lambda k: (0,)),
    ],
    out_specs=pl.BlockSpec((B, H), lambda k: (0, 0)),
    out_shape=jax.ShapeDtypeStruct((B, H), jnp.float32),
)


def kernel(x, codebook_w, enc_W, enc_b, dec_W, dec_b):
    e2d = _enc(x, enc_W, enc_b)
    ef = e2d.reshape(M, H)
    p = _prod(ef, codebook_w)
    se = jnp.sum(ef ** 2, axis=1, keepdims=True)
    sc = jnp.sum(codebook_w ** 2, axis=1)
    dist = jnp.sqrt(se + sc - 2.0 * p)
    idx = _lsa_call_sc(dist.reshape(B, OBJ, CB))
    idx_flat = idx.reshape(M)
    q = _gather(idx_flat, codebook_w.reshape(CB, 8, 128))
    q = q.reshape(B, OBJ, H)
    out = _dec(q.reshape(B, OBJ * H), dec_W, dec_b)
    e = e2d.reshape(B, OBJ, H)
    return (out, q, e)


# speedup vs baseline: 61.1849x; 1.0713x over previous
import functools

import jax
import jax.numpy as jnp
from jax.experimental import pallas as pl
from jax.experimental.pallas import tpu as pltpu

CB = 8192
H = 1024
OBJ = 32
B = 64
M = B * OBJ
G = CB // 128

_BIG = 1 << 30


def _enc_body(x_ref, w_ref, b_ref, o_ref):
    p = jax.lax.dot_general(x_ref[...], w_ref[...], (((1,), (1,)), ((), ())),
                            preferred_element_type=jnp.float32)
    o_ref[...] = p + b_ref[...][None, :]


_NT_E = 8
_BLK_E = (OBJ * H) // _NT_E

_enc = pl.pallas_call(
    _enc_body,
    grid=(_NT_E,),
    in_specs=[
        pl.BlockSpec((B, H), lambda i: (0, 0)),
        pl.BlockSpec((_BLK_E, H), lambda i: (i, 0)),
        pl.BlockSpec((_BLK_E,), lambda i: (i,)),
    ],
    out_specs=pl.BlockSpec((B, _BLK_E), lambda i: (0, i)),
    out_shape=jax.ShapeDtypeStruct((B, OBJ * H), jnp.float32),
)


def _prod_body(ef_ref, cw_ref, o_ref):
    o_ref[...] = jax.lax.dot_general(
        ef_ref[...], cw_ref[...], (((1,), (1,)), ((), ())),
        preferred_element_type=jnp.float32)


_MT, _NT = 256, 2048

_prod = pl.pallas_call(
    _prod_body,
    grid=(M // _MT, CB // _NT),
    in_specs=[
        pl.BlockSpec((_MT, H), lambda i, j: (i, 0)),
        pl.BlockSpec((_NT, H), lambda i, j: (j, 0)),
    ],
    out_specs=pl.BlockSpec((_MT, _NT), lambda i, j: (i, j)),
    out_shape=jax.ShapeDtypeStruct((M, CB), jnp.float32),
)


from jax import lax
from jax.experimental.pallas import tpu_sc as plsc

_NC = 2
_NS = 16
_NW = _NC * _NS
_SPW = B // _NW
_NCHUNK = CB // 16


def _sc_lsa_kernel(dist_hbm, out_hbm, bufa_ref, bufb_ref, gbuf_ref, v_ref,
                   sh_ref, path_ref, rem_ref, r4c_ref, c4r_ref, u_s, sr_s,
                   c4r_s, sema, semb):
    wid = lax.axis_index("s") * _NC + lax.axis_index("c")
    lane = jax.lax.broadcasted_iota(jnp.int32, (16,), 0)
    inf = jnp.float32(jnp.inf)
    inf16 = jnp.full((16,), jnp.inf, jnp.float32)
    zero16i = jnp.zeros((16,), jnp.int32)

    def ds16(c):
        return pl.ds(c * 16, 16)

    def wr16(ref, pos, val):
        sl = pl.ds((pos // 16) * 16, 16)
        ref[sl] = jnp.where(lane == (pos % 16), val, ref[sl])

    def rd16i(ref, pos):
        vec = ref[pl.ds((pos // 16) * 16, 16)]
        return jnp.sum(jnp.where(lane == (pos % 16), vec, 0))

    def rd16f(ref, pos):
        vec = ref[pl.ds((pos // 16) * 16, 16)]
        return jnp.min(jnp.where(lane == (pos % 16), vec, inf))

    for so in range(_SPW):
        sample = wid * _SPW + so
        pltpu.async_copy(dist_hbm.at[sample, 0], bufa_ref, sema)

        def init_chunk(c, carry):
            v_ref[ds16(c)] = jnp.zeros((16,), jnp.float32)
            r4c_ref[ds16(c)] = jnp.full((16,), -1, jnp.int32)
            return carry

        lax.fori_loop(0, _NCHUNK, init_chunk, 0, unroll=8)

        def init_small(rr, carry):
            u_s[rr] = jnp.float32(0.0)
            c4r_s[rr] = jnp.int32(-1)
            return carry

        lax.fori_loop(0, OBJ, init_small, 0)
        c4r_ref[pl.ds(0, 16)] = jnp.full((16,), -1, jnp.int32)
        c4r_ref[pl.ds(16, 16)] = jnp.full((16,), -1, jnp.int32)

        def row_body(cur_row, crow_ref, sem, obuf_ref, osem):
            pltpu.make_async_copy(
                dist_hbm.at[sample, cur_row], crow_ref, sem).wait()

            @pl.when(cur_row + 1 < OBJ)
            def _():
                pltpu.async_copy(
                    dist_hbm.at[sample, cur_row + 1], obuf_ref, osem)

            def fp_chunk(c, st):
                rm, rc = st
                x = crow_ref[ds16(c)] - v_ref[ds16(c)]
                lt = x < rm
                rm = jnp.where(lt, x, rm)
                rc = jnp.where(lt, c, rc)
                return (rm, rc)

            rm, rc = lax.fori_loop(0, _NCHUNK, fp_chunk, (inf16, zero16i),
                                   unroll=8)
            mv0 = jnp.min(rm)
            j0 = jnp.min(jnp.where(rm == mv0, rc * 16 + lane, _BIG))
            taken = rd16i(r4c_ref, j0) != -1

            @pl.when(jnp.logical_not(taken))
            def _():
                u_s[cur_row] = u_s[cur_row] + mv0
                c4r_s[cur_row] = j0
                wr16(c4r_ref, cur_row, j0)
                wr16(r4c_ref, j0, cur_row)

            @pl.when(taken)
            def _():
                def ginit(c, carryg):
                    sh_ref[ds16(c)] = inf16
                    path_ref[ds16(c)] = jnp.full((16,), -1, jnp.int32)
                    rem_ref[ds16(c)] = jnp.full((16,), 1, jnp.int32)
                    return carryg

                lax.fori_loop(0, _NCHUNK, ginit, 0, unroll=8)

                def clear_sr(rr, carryg):
                    sr_s[rr] = jnp.int32(0)
                    return carryg

                lax.fori_loop(0, OBJ, clear_sr, 0)

                def search_cond(s):
                    return s[1] == -1

                def search_body(s):
                    i, sink, min_val, nsr = s
                    sr_s[i] = jnp.int32(1)
                    pltpu.sync_copy(dist_hbm.at[sample, i], gbuf_ref)
                    u_i = u_s[i]

                    def schunk(c, st):
                        srm, src_ = st
                        x = ((min_val + gbuf_ref[ds16(c)]) - u_i) - v_ref[ds16(c)]
                        sh = sh_ref[ds16(c)]
                        remc = rem_ref[ds16(c)]
                        remb = remc != 0
                        upd = remb & (x < sh)
                        sh2 = jnp.where(upd, x, sh)
                        sh_ref[ds16(c)] = sh2
                        path_ref[ds16(c)] = jnp.where(upd, i, path_ref[ds16(c)])
                        masked = jnp.where(remb, sh2, inf)
                        lt = masked < srm
                        srm = jnp.where(lt, masked, srm)
                        src_ = jnp.where(lt, c, src_)
                        return (srm, src_)

                    srm, src_ = lax.fori_loop(0, _NCHUNK, schunk,
                                              (inf16, zero16i), unroll=4)
                    mv = jnp.min(srm)
                    j = jnp.min(jnp.where(srm == mv, src_ * 16 + lane, _BIG))
                    wr16(rem_ref, j, jnp.int32(0))
                    r4c_j = rd16i(r4c_ref, j)
                    hit = r4c_j == -1
                    sink = jnp.where(hit, j, sink)
                    i = jnp.where(hit, i, r4c_j)
                    return (i, sink, mv, nsr + 1)

                init = (jnp.int32(0) + cur_row, jnp.int32(-1),
                        jnp.float32(0.0), jnp.int32(0))
                _, sink, min_val, nsr = lax.while_loop(
                    search_cond, search_body, init)

                u_s[cur_row] = u_s[cur_row] + min_val

                @pl.when(nsr > 1)
                def _():
                    def ex_body(rr, carryg):
                        take = (sr_s[rr] > 0) & (rr != cur_row)

                        @pl.when(take)
                        def _():
                            s_val = rd16f(sh_ref, c4r_s[rr])
                            u_s[rr] = u_s[rr] + (min_val - s_val)

                        return carryg

                    lax.fori_loop(0, OBJ, ex_body, 0)

                def vchunk(c, carryg):
                    t = min_val - sh_ref[ds16(c)]
                    remc = rem_ref[ds16(c)]
                    v_ref[ds16(c)] = jnp.where(remc == 0,
                                               v_ref[ds16(c)] - t,
                                               v_ref[ds16(c)])
                    return carryg

                lax.fori_loop(0, _NCHUNK, vchunk, 0, unroll=4)

                def aug_cond(s):
                    return jnp.logical_not(s[1])

                def aug_body(s):
                    j, done = s
                    i = rd16i(path_ref, j)
                    wr16(r4c_ref, j, i)
                    nj = c4r_s[i]
                    c4r_s[i] = j
                    wr16(c4r_ref, i, j)
                    done = i == cur_row
                    return (nj, done)

                lax.while_loop(aug_cond, aug_body, (sink, jnp.bool_(False)))

        def outer(kk, carry):
            row_body(kk * 2, bufa_ref, sema, bufb_ref, semb)
            row_body(kk * 2 + 1, bufb_ref, semb, bufa_ref, sema)
            return carry

        lax.fori_loop(0, OBJ // 2, outer, 0)
        pltpu.sync_copy(c4r_ref, out_hbm.at[sample])


_lsa_call_sc = functools.partial(
    pl.kernel,
    mesh=plsc.VectorSubcoreMesh(core_axis_name="c", subcore_axis_name="s"),
    compiler_params=pltpu.CompilerParams(needs_layout_passes=False),
    out_type=jax.ShapeDtypeStruct((B, OBJ), jnp.int32),
    scratch_types=[
        pltpu.VMEM((CB,), jnp.float32),
        pltpu.VMEM((CB,), jnp.float32),
        pltpu.VMEM((CB,), jnp.float32),
        pltpu.VMEM((CB,), jnp.float32),
        pltpu.VMEM((CB,), jnp.float32),
        pltpu.VMEM((CB,), jnp.int32),
        pltpu.VMEM((CB,), jnp.int32),
        pltpu.VMEM((CB,), jnp.int32),
        pltpu.VMEM((OBJ,), jnp.int32),
        pltpu.SMEM((OBJ,), jnp.float32),
        pltpu.SMEM((OBJ,), jnp.int32),
        pltpu.SMEM((OBJ,), jnp.int32),
        pltpu.SemaphoreType.DMA,
        pltpu.SemaphoreType.DMA,
    ],
)(_sc_lsa_kernel)


def _gather_body(idx_ref, cw_ref, o_ref):
    s = pl.program_id(0)

    def body(r, _):
        o_ref[0, r] = cw_ref[idx_ref[s * OBJ + r]]
        return 0

    jax.lax.fori_loop(0, OBJ, body, 0)


_gather = pl.pallas_call(
    _gather_body,
    grid_spec=pltpu.PrefetchScalarGridSpec(
        num_scalar_prefetch=1,
        grid=(B,),
        in_specs=[pl.BlockSpec((CB, 8, 128), lambda s, idx: (0, 0, 0))],
        out_specs=pl.BlockSpec((1, OBJ, 8, 128), lambda s, idx: (s, 0, 0, 0)),
    ),
    out_shape=jax.ShapeDtypeStruct((B, OBJ, 8, 128), jnp.float32),
)


def _dec_body(q_ref, w_ref, b_ref, o_ref):
    k = pl.program_id(0)
    nk = pl.num_programs(0)
    p = jax.lax.dot_general(q_ref[...], w_ref[...], (((1,), (1,)), ((), ())),
                            preferred_element_type=jnp.float32)

    @pl.when(k == 0)
    def _():
        o_ref[...] = p

    @pl.when(k > 0)
    def _():
        o_ref[...] += p

    @pl.when(k == nk - 1)
    def _():
        o_ref[...] += b_ref[...][None, :]


_NK_D = 8
_BLK_D = (OBJ * H) // _NK_D

_dec = pl.pallas_call(
    _dec_body,
    grid=(_NK_D,),
    in_specs=[
        pl.BlockSpec((B, _BLK_D), lambda k: (0, k)),
        pl.BlockSpec((H, _BLK_D), lambda k: (0, k)),
        pl.BlockSpec((H,), lambda k: (0,)),
    ],
    out_specs=pl.BlockSpec((B, H), lambda k: (0, 0)),
    out_shape=jax.ShapeDtypeStruct((B, H), jnp.float32),
)


def kernel(x, codebook_w, enc_W, enc_b, dec_W, dec_b):
    e2d = _enc(x, enc_W, enc_b)
    ef = e2d.reshape(M, H)
    p = _prod(ef, codebook_w)
    se = jnp.sum(ef ** 2, axis=1, keepdims=True)
    sc = jnp.sum(codebook_w ** 2, axis=1)
    dist = jnp.sqrt(se + sc - 2.0 * p)
    idx = _lsa_call_sc(dist.reshape(B, OBJ, CB))
    idx_flat = idx.reshape(M)
    q = _gather(idx_flat, codebook_w.reshape(CB, 8, 128))
    q = q.reshape(B, OBJ, H)
    out = _dec(q.reshape(B, OBJ * H), dec_W, dec_b)
    e = e2d.reshape(B, OBJ, H)
    return (out, q, e)


# speedup vs baseline: 68.2442x; 1.1154x over previous
import functools

import jax
import jax.numpy as jnp
from jax.experimental import pallas as pl
from jax.experimental.pallas import tpu as pltpu

CB = 8192
H = 1024
OBJ = 32
B = 64
M = B * OBJ
G = CB // 128

_BIG = 1 << 30


def _enc_body(x_ref, w_ref, b_ref, o_ref):
    p = jax.lax.dot_general(x_ref[...], w_ref[...], (((1,), (1,)), ((), ())),
                            preferred_element_type=jnp.float32)
    o_ref[...] = p + b_ref[...][None, :]


_NT_E = 8
_BLK_E = (OBJ * H) // _NT_E

_enc = pl.pallas_call(
    _enc_body,
    grid=(_NT_E,),
    in_specs=[
        pl.BlockSpec((B, H), lambda i: (0, 0)),
        pl.BlockSpec((_BLK_E, H), lambda i: (i, 0)),
        pl.BlockSpec((_BLK_E,), lambda i: (i,)),
    ],
    out_specs=pl.BlockSpec((B, _BLK_E), lambda i: (0, i)),
    out_shape=jax.ShapeDtypeStruct((B, OBJ * H), jnp.float32),
)


def _prod_body(ef_ref, cw_ref, o_ref):
    o_ref[...] = jax.lax.dot_general(
        ef_ref[...], cw_ref[...], (((1,), (1,)), ((), ())),
        preferred_element_type=jnp.float32)


_MT, _NT = 256, 2048

_prod = pl.pallas_call(
    _prod_body,
    grid=(M // _MT, CB // _NT),
    in_specs=[
        pl.BlockSpec((_MT, H), lambda i, j: (i, 0)),
        pl.BlockSpec((_NT, H), lambda i, j: (j, 0)),
    ],
    out_specs=pl.BlockSpec((_MT, _NT), lambda i, j: (i, j)),
    out_shape=jax.ShapeDtypeStruct((M, CB), jnp.float32),
)


from jax import lax
from jax.experimental.pallas import tpu_sc as plsc

_NC = 2
_NS = 16
_NW = _NC * _NS
_SPW = B // _NW
_NCHUNK = CB // 16


def _sc_lsa_kernel(dist_hbm, cb_hbm, out_hbm, q_hbm, bufa_ref, bufb_ref, gbuf_ref, v_ref,
                   sh_ref, path_ref, rem_ref, r4c_ref, c4r_ref, u_s, sr_s,
                   qrows_ref, c4r_s, vnz_s, sema, semb, semq):
    wid = lax.axis_index("s") * _NC + lax.axis_index("c")
    lane = jax.lax.broadcasted_iota(jnp.int32, (16,), 0)
    inf = jnp.float32(jnp.inf)
    inf16 = jnp.full((16,), jnp.inf, jnp.float32)
    zero16i = jnp.zeros((16,), jnp.int32)

    def ds16(c):
        return pl.ds(c * 16, 16)

    def wr16(ref, pos, val):
        sl = pl.ds((pos // 16) * 16, 16)
        ref[sl] = jnp.where(lane == (pos % 16), val, ref[sl])

    def rd16i(ref, pos):
        vec = ref[pl.ds((pos // 16) * 16, 16)]
        return jnp.sum(jnp.where(lane == (pos % 16), vec, 0))

    def rd16f(ref, pos):
        vec = ref[pl.ds((pos // 16) * 16, 16)]
        return jnp.min(jnp.where(lane == (pos % 16), vec, inf))

    for so in range(_SPW):
        sample = wid * _SPW + so
        pltpu.async_copy(dist_hbm.at[sample * OBJ], bufa_ref, sema)

        def init_chunk(c, carry):
            v_ref[ds16(c)] = jnp.zeros((16,), jnp.float32)
            r4c_ref[ds16(c)] = jnp.full((16,), -1, jnp.int32)
            return carry

        lax.fori_loop(0, _NCHUNK, init_chunk, 0, unroll=8)

        def init_small(rr, carry):
            u_s[rr] = jnp.float32(0.0)
            c4r_s[rr] = jnp.int32(-1)
            return carry

        lax.fori_loop(0, OBJ, init_small, 0)
        vnz_s[0] = jnp.int32(0)
        c4r_ref[pl.ds(0, 16)] = jnp.full((16,), -1, jnp.int32)
        c4r_ref[pl.ds(16, 16)] = jnp.full((16,), -1, jnp.int32)

        def row_body(cur_row, crow_ref, sem, obuf_ref, osem):
            pltpu.make_async_copy(
                dist_hbm.at[sample * OBJ + cur_row], crow_ref, sem).wait()

            @pl.when(cur_row + 1 < OBJ)
            def _():
                pltpu.async_copy(
                    dist_hbm.at[sample * OBJ + cur_row + 1], obuf_ref, osem)

            def fp_chunk(c, st):
                rm, rc = st
                x = crow_ref[ds16(c)] - v_ref[ds16(c)]
                lt = x < rm
                rm = jnp.where(lt, x, rm)
                rc = jnp.where(lt, c, rc)
                return (rm, rc)

            def fp_chunk_vz(c, st):
                rm, rc = st
                x = crow_ref[ds16(c)]
                lt = x < rm
                rm = jnp.where(lt, x, rm)
                rc = jnp.where(lt, c, rc)
                return (rm, rc)

            rm, rc = lax.cond(
                vnz_s[0] == 0,
                lambda: lax.fori_loop(0, _NCHUNK, fp_chunk_vz,
                                      (inf16, zero16i), unroll=8),
                lambda: lax.fori_loop(0, _NCHUNK, fp_chunk,
                                      (inf16, zero16i), unroll=8))
            mv0 = jnp.min(rm)
            j0 = jnp.min(jnp.where(rm == mv0, rc * 16 + lane, _BIG))
            taken = rd16i(r4c_ref, j0) != -1

            @pl.when(jnp.logical_not(taken))
            def _():
                u_s[cur_row] = u_s[cur_row] + mv0
                c4r_s[cur_row] = j0
                wr16(c4r_ref, cur_row, j0)
                wr16(r4c_ref, j0, cur_row)

            @pl.when(taken)
            def _():
                def ginit(c, carryg):
                    sh_ref[ds16(c)] = inf16
                    path_ref[ds16(c)] = jnp.full((16,), -1, jnp.int32)
                    rem_ref[ds16(c)] = jnp.full((16,), 1, jnp.int32)
                    return carryg

                lax.fori_loop(0, _NCHUNK, ginit, 0, unroll=8)

                def clear_sr(rr, carryg):
                    sr_s[rr] = jnp.int32(0)
                    return carryg

                lax.fori_loop(0, OBJ, clear_sr, 0)

                def search_cond(s):
                    return s[1] == -1

                def search_body(s):
                    i, sink, min_val, nsr = s
                    sr_s[i] = jnp.int32(1)
                    pltpu.sync_copy(dist_hbm.at[sample * OBJ + i], gbuf_ref)
                    u_i = u_s[i]

                    def schunk(c, st):
                        srm, src_ = st
                        x = ((min_val + gbuf_ref[ds16(c)]) - u_i) - v_ref[ds16(c)]
                        sh = sh_ref[ds16(c)]
                        remc = rem_ref[ds16(c)]
                        remb = remc != 0
                        upd = remb & (x < sh)
                        sh2 = jnp.where(upd, x, sh)
                        sh_ref[ds16(c)] = sh2
                        path_ref[ds16(c)] = jnp.where(upd, i, path_ref[ds16(c)])
                        masked = jnp.where(remb, sh2, inf)
                        lt = masked < srm
                        srm = jnp.where(lt, masked, srm)
                        src_ = jnp.where(lt, c, src_)
                        return (srm, src_)

                    srm, src_ = lax.fori_loop(0, _NCHUNK, schunk,
                                              (inf16, zero16i), unroll=4)
                    mv = jnp.min(srm)
                    j = jnp.min(jnp.where(srm == mv, src_ * 16 + lane, _BIG))
                    wr16(rem_ref, j, jnp.int32(0))
                    r4c_j = rd16i(r4c_ref, j)
                    hit = r4c_j == -1
                    sink = jnp.where(hit, j, sink)
                    i = jnp.where(hit, i, r4c_j)
                    return (i, sink, mv, nsr + 1)

                init = (jnp.int32(0) + cur_row, jnp.int32(-1),
                        jnp.float32(0.0), jnp.int32(0))
                _, sink, min_val, nsr = lax.while_loop(
                    search_cond, search_body, init)

                u_s[cur_row] = u_s[cur_row] + min_val

                @pl.when(nsr > 1)
                def _():
                    vnz_s[0] = jnp.int32(1)

                @pl.when(nsr > 1)
                def _():
                    def ex_body(rr, carryg):
                        take = (sr_s[rr] > 0) & (rr != cur_row)

                        @pl.when(take)
                        def _():
                            s_val = rd16f(sh_ref, c4r_s[rr])
                            u_s[rr] = u_s[rr] + (min_val - s_val)

                        return carryg

                    lax.fori_loop(0, OBJ, ex_body, 0)

                def vchunk(c, carryg):
                    t = min_val - sh_ref[ds16(c)]
                    remc = rem_ref[ds16(c)]
                    v_ref[ds16(c)] = jnp.where(remc == 0,
                                               v_ref[ds16(c)] - t,
                                               v_ref[ds16(c)])
                    return carryg

                lax.fori_loop(0, _NCHUNK, vchunk, 0, unroll=4)

                def aug_cond(s):
                    return jnp.logical_not(s[1])

                def aug_body(s):
                    j, done = s
                    i = rd16i(path_ref, j)
                    wr16(r4c_ref, j, i)
                    nj = c4r_s[i]
                    c4r_s[i] = j
                    wr16(c4r_ref, i, j)
                    done = i == cur_row
                    return (nj, done)

                lax.while_loop(aug_cond, aug_body, (sink, jnp.bool_(False)))

        def outer(kk, carry):
            row_body(kk * 2, bufa_ref, sema, bufb_ref, semb)
            row_body(kk * 2 + 1, bufb_ref, semb, bufa_ref, sema)
            return carry

        lax.fori_loop(0, OBJ // 2, outer, 0)
        pltpu.sync_copy(c4r_ref, out_hbm.at[sample])
        pltpu.async_copy(cb_hbm.at[c4r_ref], qrows_ref, semq).wait()
        pltpu.sync_copy(qrows_ref, q_hbm.at[sample])


_lsa_call_sc = functools.partial(
    pl.kernel,
    mesh=plsc.VectorSubcoreMesh(core_axis_name="c", subcore_axis_name="s"),
    compiler_params=pltpu.CompilerParams(needs_layout_passes=False),
    out_type=[jax.ShapeDtypeStruct((B, OBJ), jnp.int32),
              jax.ShapeDtypeStruct((B, OBJ, H), jnp.float32)],
    scratch_types=[
        pltpu.VMEM((CB,), jnp.float32),
        pltpu.VMEM((CB,), jnp.float32),
        pltpu.VMEM((CB,), jnp.float32),
        pltpu.VMEM((CB,), jnp.float32),
        pltpu.VMEM((CB,), jnp.float32),
        pltpu.VMEM((CB,), jnp.int32),
        pltpu.VMEM((CB,), jnp.int32),
        pltpu.VMEM((CB,), jnp.int32),
        pltpu.VMEM((OBJ,), jnp.int32),
        pltpu.SMEM((OBJ,), jnp.float32),
        pltpu.SMEM((OBJ,), jnp.int32),
        pltpu.VMEM((OBJ, H), jnp.float32),
        pltpu.SMEM((OBJ,), jnp.int32),
        pltpu.SMEM((1,), jnp.int32),
        pltpu.SemaphoreType.DMA,
        pltpu.SemaphoreType.DMA,
        pltpu.SemaphoreType.DMA,
    ],
)(_sc_lsa_kernel)


def _gather_body(idx_ref, cw_ref, o_ref):
    s = pl.program_id(0)

    def body(r, _):
        o_ref[0, r] = cw_ref[idx_ref[s * OBJ + r]]
        return 0

    jax.lax.fori_loop(0, OBJ, body, 0)


_gather = pl.pallas_call(
    _gather_body,
    grid_spec=pltpu.PrefetchScalarGridSpec(
        num_scalar_prefetch=1,
        grid=(B,),
        in_specs=[pl.BlockSpec((CB, 8, 128), lambda s, idx: (0, 0, 0))],
        out_specs=pl.BlockSpec((1, OBJ, 8, 128), lambda s, idx: (s, 0, 0, 0)),
    ),
    out_shape=jax.ShapeDtypeStruct((B, OBJ, 8, 128), jnp.float32),
)


def _dec_body(q_ref, w_ref, b_ref, o_ref):
    k = pl.program_id(0)
    nk = pl.num_programs(0)
    p = jax.lax.dot_general(q_ref[...], w_ref[...], (((1,), (1,)), ((), ())),
                            preferred_element_type=jnp.float32)

    @pl.when(k == 0)
    def _():
        o_ref[...] = p

    @pl.when(k > 0)
    def _():
        o_ref[...] += p

    @pl.when(k == nk - 1)
    def _():
        o_ref[...] += b_ref[...][None, :]


_NK_D = 8
_BLK_D = (OBJ * H) // _NK_D

_dec = pl.pallas_call(
    _dec_body,
    grid=(_NK_D,),
    in_specs=[
        pl.BlockSpec((B, _BLK_D), lambda k: (0, k)),
        pl.BlockSpec((H, _BLK_D), lambda k: (0, k)),
        pl.BlockSpec((H,), lambda k: (0,)),
    ],
    out_specs=pl.BlockSpec((B, H), lambda k: (0, 0)),
    out_shape=jax.ShapeDtypeStruct((B, H), jnp.float32),
)


def kernel(x, codebook_w, enc_W, enc_b, dec_W, dec_b):
    e2d = _enc(x, enc_W, enc_b)
    ef = e2d.reshape(M, H)
    p = _prod(ef, codebook_w)
    se = jnp.sum(ef ** 2, axis=1, keepdims=True)
    sc = jnp.sum(codebook_w ** 2, axis=1)
    dist = jnp.sqrt(se + sc - 2.0 * p)
    _, q = _lsa_call_sc(dist, codebook_w)
    out = _dec(q.reshape(B, OBJ * H), dec_W, dec_b)
    e = e2d.reshape(B, OBJ, H)
    return (out, q, e)


# speedup vs baseline: 72.6657x; 1.0648x over previous
import functools

import jax
import jax.numpy as jnp
from jax.experimental import pallas as pl
from jax.experimental.pallas import tpu as pltpu

CB = 8192
H = 1024
OBJ = 32
B = 64
M = B * OBJ
G = CB // 128

_BIG = 1 << 30


def _enc_body(x_ref, w_ref, b_ref, o_ref):
    p = jax.lax.dot_general(x_ref[...], w_ref[...], (((1,), (1,)), ((), ())),
                            preferred_element_type=jnp.float32)
    o_ref[...] = p + b_ref[...][None, :]


_NT_E = 8
_BLK_E = (OBJ * H) // _NT_E

_enc = pl.pallas_call(
    _enc_body,
    grid=(_NT_E,),
    in_specs=[
        pl.BlockSpec((B, H), lambda i: (0, 0)),
        pl.BlockSpec((_BLK_E, H), lambda i: (i, 0)),
        pl.BlockSpec((_BLK_E,), lambda i: (i,)),
    ],
    out_specs=pl.BlockSpec((B, _BLK_E), lambda i: (0, i)),
    out_shape=jax.ShapeDtypeStruct((B, OBJ * H), jnp.float32),
)


def _prod_body(ef_ref, cw_ref, o_ref):
    o_ref[...] = jax.lax.dot_general(
        ef_ref[...], cw_ref[...], (((1,), (1,)), ((), ())),
        preferred_element_type=jnp.float32)


_MT, _NT = 512, 2048

_prod = pl.pallas_call(
    _prod_body,
    grid=(M // _MT, CB // _NT),
    in_specs=[
        pl.BlockSpec((_MT, H), lambda i, j: (i, 0)),
        pl.BlockSpec((_NT, H), lambda i, j: (j, 0)),
    ],
    out_specs=pl.BlockSpec((_MT, _NT), lambda i, j: (i, j)),
    out_shape=jax.ShapeDtypeStruct((M, CB), jnp.float32),
)


from jax import lax
from jax.experimental.pallas import tpu_sc as plsc

_NC = 2
_NS = 16
_NW = _NC * _NS
_SPW = B // _NW
_NCHUNK = CB // 16


def _sc_lsa_kernel(dist_hbm, cb_hbm, out_hbm, q_hbm, bufa_ref, bufb_ref, gbuf_ref, v_ref,
                   sh_ref, path_ref, rem_ref, r4c_ref, c4r_ref, u_s, sr_s,
                   qrows_ref, c4r_s, vnz_s, sema, semb, semq):
    wid = lax.axis_index("s") * _NC + lax.axis_index("c")
    lane = jax.lax.broadcasted_iota(jnp.int32, (16,), 0)
    inf = jnp.float32(jnp.inf)
    inf16 = jnp.full((16,), jnp.inf, jnp.float32)
    zero16i = jnp.zeros((16,), jnp.int32)

    def ds16(c):
        return pl.ds(c * 16, 16)

    def wr16(ref, pos, val):
        sl = pl.ds((pos // 16) * 16, 16)
        ref[sl] = jnp.where(lane == (pos % 16), val, ref[sl])

    def rd16i(ref, pos):
        vec = ref[pl.ds((pos // 16) * 16, 16)]
        return jnp.sum(jnp.where(lane == (pos % 16), vec, 0))

    def rd16f(ref, pos):
        vec = ref[pl.ds((pos // 16) * 16, 16)]
        return jnp.min(jnp.where(lane == (pos % 16), vec, inf))

    for so in range(_SPW):
        sample = wid * _SPW + so
        pltpu.async_copy(dist_hbm.at[sample * OBJ], bufa_ref, sema)

        def init_chunk(c, carry):
            v_ref[ds16(c)] = jnp.zeros((16,), jnp.float32)
            r4c_ref[ds16(c)] = jnp.full((16,), -1, jnp.int32)
            return carry

        lax.fori_loop(0, _NCHUNK, init_chunk, 0, unroll=8)

        def init_small(rr, carry):
            u_s[rr] = jnp.float32(0.0)
            c4r_s[rr] = jnp.int32(-1)
            return carry

        lax.fori_loop(0, OBJ, init_small, 0)
        vnz_s[0] = jnp.int32(0)
        c4r_ref[pl.ds(0, 16)] = jnp.full((16,), -1, jnp.int32)
        c4r_ref[pl.ds(16, 16)] = jnp.full((16,), -1, jnp.int32)

        def row_body(cur_row, crow_ref, sem, obuf_ref, osem):
            pltpu.make_async_copy(
                dist_hbm.at[sample * OBJ + cur_row], crow_ref, sem).wait()

            @pl.when(cur_row + 1 < OBJ)
            def _():
                pltpu.async_copy(
                    dist_hbm.at[sample * OBJ + cur_row + 1], obuf_ref, osem)

            def fp_chunk(c, st):
                rm, rc = st
                x = crow_ref[ds16(c)] - v_ref[ds16(c)]
                lt = x < rm
                rm = jnp.where(lt, x, rm)
                rc = jnp.where(lt, c, rc)
                return (rm, rc)

            def fp_chunk_vz(c, st):
                rm, rc = st
                x = crow_ref[ds16(c)]
                lt = x < rm
                rm = jnp.where(lt, x, rm)
                rc = jnp.where(lt, c, rc)
                return (rm, rc)

            rm, rc = lax.cond(
                vnz_s[0] == 0,
                lambda: lax.fori_loop(0, _NCHUNK, fp_chunk_vz,
                                      (inf16, zero16i), unroll=16),
                lambda: lax.fori_loop(0, _NCHUNK, fp_chunk,
                                      (inf16, zero16i), unroll=16))
            mv0 = jnp.min(rm)
            j0 = jnp.min(jnp.where(rm == mv0, rc * 16 + lane, _BIG))
            taken = rd16i(r4c_ref, j0) != -1

            @pl.when(jnp.logical_not(taken))
            def _():
                u_s[cur_row] = u_s[cur_row] + mv0
                c4r_s[cur_row] = j0
                wr16(c4r_ref, cur_row, j0)
                wr16(r4c_ref, j0, cur_row)

            @pl.when(taken)
            def _():
                def ginit(c, carryg):
                    sh_ref[ds16(c)] = inf16
                    path_ref[ds16(c)] = jnp.full((16,), -1, jnp.int32)
                    rem_ref[ds16(c)] = jnp.full((16,), 1, jnp.int32)
                    return carryg

                lax.fori_loop(0, _NCHUNK, ginit, 0, unroll=8)

                def clear_sr(rr, carryg):
                    sr_s[rr] = jnp.int32(0)
                    return carryg

                lax.fori_loop(0, OBJ, clear_sr, 0)

                def search_cond(s):
                    return s[1] == -1

                def search_body(s):
                    i, sink, min_val, nsr = s
                    sr_s[i] = jnp.int32(1)
                    pltpu.sync_copy(dist_hbm.at[sample * OBJ + i], gbuf_ref)
                    u_i = u_s[i]

                    def schunk(c, st):
                        srm, src_ = st
                        x = ((min_val + gbuf_ref[ds16(c)]) - u_i) - v_ref[ds16(c)]
                        sh = sh_ref[ds16(c)]
                        remc = rem_ref[ds16(c)]
                        remb = remc != 0
                        upd = remb & (x < sh)
                        sh2 = jnp.where(upd, x, sh)
                        sh_ref[ds16(c)] = sh2
                        path_ref[ds16(c)] = jnp.where(upd, i, path_ref[ds16(c)])
                        masked = jnp.where(remb, sh2, inf)
                        lt = masked < srm
                        srm = jnp.where(lt, masked, srm)
                        src_ = jnp.where(lt, c, src_)
                        return (srm, src_)

                    srm, src_ = lax.fori_loop(0, _NCHUNK, schunk,
                                              (inf16, zero16i), unroll=8)
                    mv = jnp.min(srm)
                    j = jnp.min(jnp.where(srm == mv, src_ * 16 + lane, _BIG))
                    wr16(rem_ref, j, jnp.int32(0))
                    r4c_j = rd16i(r4c_ref, j)
                    hit = r4c_j == -1
                    sink = jnp.where(hit, j, sink)
                    i = jnp.where(hit, i, r4c_j)
                    return (i, sink, mv, nsr + 1)

                init = (jnp.int32(0) + cur_row, jnp.int32(-1),
                        jnp.float32(0.0), jnp.int32(0))
                _, sink, min_val, nsr = lax.while_loop(
                    search_cond, search_body, init)

                u_s[cur_row] = u_s[cur_row] + min_val

                @pl.when(nsr > 1)
                def _():
                    vnz_s[0] = jnp.int32(1)

                @pl.when(nsr > 1)
                def _():
                    def ex_body(rr, carryg):
                        take = (sr_s[rr] > 0) & (rr != cur_row)

                        @pl.when(take)
                        def _():
                            s_val = rd16f(sh_ref, c4r_s[rr])
                            u_s[rr] = u_s[rr] + (min_val - s_val)

                        return carryg

                    lax.fori_loop(0, OBJ, ex_body, 0)

                def vchunk(c, carryg):
                    t = min_val - sh_ref[ds16(c)]
                    remc = rem_ref[ds16(c)]
                    v_ref[ds16(c)] = jnp.where(remc == 0,
                                               v_ref[ds16(c)] - t,
                                               v_ref[ds16(c)])
                    return carryg

                lax.fori_loop(0, _NCHUNK, vchunk, 0, unroll=4)

                def aug_cond(s):
                    return jnp.logical_not(s[1])

                def aug_body(s):
                    j, done = s
                    i = rd16i(path_ref, j)
                    wr16(r4c_ref, j, i)
                    nj = c4r_s[i]
                    c4r_s[i] = j
                    wr16(c4r_ref, i, j)
                    done = i == cur_row
                    return (nj, done)

                lax.while_loop(aug_cond, aug_body, (sink, jnp.bool_(False)))

        def outer(kk, carry):
            row_body(kk * 2, bufa_ref, sema, bufb_ref, semb)
            row_body(kk * 2 + 1, bufb_ref, semb, bufa_ref, sema)
            return carry

        lax.fori_loop(0, OBJ // 2, outer, 0)
        pltpu.sync_copy(c4r_ref, out_hbm.at[sample])
        pltpu.async_copy(cb_hbm.at[c4r_ref], qrows_ref, semq).wait()
        pltpu.sync_copy(qrows_ref, q_hbm.at[sample])


_lsa_call_sc = functools.partial(
    pl.kernel,
    mesh=plsc.VectorSubcoreMesh(core_axis_name="c", subcore_axis_name="s"),
    compiler_params=pltpu.CompilerParams(needs_layout_passes=False),
    out_type=[jax.ShapeDtypeStruct((B, OBJ), jnp.int32),
              jax.ShapeDtypeStruct((B, OBJ, H), jnp.float32)],
    scratch_types=[
        pltpu.VMEM((CB,), jnp.float32),
        pltpu.VMEM((CB,), jnp.float32),
        pltpu.VMEM((CB,), jnp.float32),
        pltpu.VMEM((CB,), jnp.float32),
        pltpu.VMEM((CB,), jnp.float32),
        pltpu.VMEM((CB,), jnp.int32),
        pltpu.VMEM((CB,), jnp.int32),
        pltpu.VMEM((CB,), jnp.int32),
        pltpu.VMEM((OBJ,), jnp.int32),
        pltpu.SMEM((OBJ,), jnp.float32),
        pltpu.SMEM((OBJ,), jnp.int32),
        pltpu.VMEM((OBJ, H), jnp.float32),
        pltpu.SMEM((OBJ,), jnp.int32),
        pltpu.SMEM((1,), jnp.int32),
        pltpu.SemaphoreType.DMA,
        pltpu.SemaphoreType.DMA,
        pltpu.SemaphoreType.DMA,
    ],
)(_sc_lsa_kernel)


def _dec_body(q_ref, w_ref, b_ref, o_ref):
    k = pl.program_id(0)
    nk = pl.num_programs(0)
    p = jax.lax.dot_general(q_ref[...], w_ref[...], (((1,), (1,)), ((), ())),
                            preferred_element_type=jnp.float32)

    @pl.when(k == 0)
    def _():
        o_ref[...] = p

    @pl.when(k > 0)
    def _():
        o_ref[...] += p

    @pl.when(k == nk - 1)
    def _():
        o_ref[...] += b_ref[...][None, :]


_NK_D = 8
_BLK_D = (OBJ * H) // _NK_D

_dec = pl.pallas_call(
    _dec_body,
    grid=(_NK_D,),
    in_specs=[
        pl.BlockSpec((B, _BLK_D), lambda k: (0, k)),
        pl.BlockSpec((H, _BLK_D), lambda k: (0, k)),
        pl.BlockSpec((H,), lambda k: (0,)),
    ],
    out_specs=pl.BlockSpec((B, H), lambda k: (0, 0)),
    out_shape=jax.ShapeDtypeStruct((B, H), jnp.float32),
)


def kernel(x, codebook_w, enc_W, enc_b, dec_W, dec_b):
    e2d = _enc(x, enc_W, enc_b)
    ef = e2d.reshape(M, H)
    p = _prod(ef, codebook_w)
    se = jnp.sum(ef ** 2, axis=1, keepdims=True)
    sc = jnp.sum(codebook_w ** 2, axis=1)
    dist = jnp.sqrt(se + sc - 2.0 * p)
    _, q = _lsa_call_sc(dist, codebook_w)
    out = _dec(q.reshape(B, OBJ * H), dec_W, dec_b)
    e = e2d.reshape(B, OBJ, H)
    return (out, q, e)


# speedup vs baseline: 81.8934x; 1.1270x over previous
import functools

import jax
import jax.numpy as jnp
from jax.experimental import pallas as pl
from jax.experimental.pallas import tpu as pltpu

CB = 8192
H = 1024
OBJ = 32
B = 64
M = B * OBJ
G = CB // 128

_BIG = 1 << 30


def _enc_body(x_ref, w_ref, b_ref, o_ref):
    p = jax.lax.dot_general(x_ref[...], w_ref[...], (((1,), (1,)), ((), ())),
                            preferred_element_type=jnp.float32)
    o_ref[...] = p + b_ref[...][None, :]


_NT_E = 8
_BLK_E = (OBJ * H) // _NT_E

_enc = pl.pallas_call(
    _enc_body,
    grid=(_NT_E,),
    in_specs=[
        pl.BlockSpec((B, H), lambda i: (0, 0)),
        pl.BlockSpec((_BLK_E, H), lambda i: (i, 0)),
        pl.BlockSpec((_BLK_E,), lambda i: (i,)),
    ],
    out_specs=pl.BlockSpec((B, _BLK_E), lambda i: (0, i)),
    out_shape=jax.ShapeDtypeStruct((B, OBJ * H), jnp.float32),
)


def _prod_body(ef_ref, cw_ref, o_ref):
    o_ref[...] = jax.lax.dot_general(
        ef_ref[...], cw_ref[...], (((1,), (1,)), ((), ())),
        preferred_element_type=jnp.float32)


_MT, _NT = 1024, 2048

_prod = pl.pallas_call(
    _prod_body,
    grid=(M // _MT, CB // _NT),
    in_specs=[
        pl.BlockSpec((_MT, H), lambda i, j: (i, 0)),
        pl.BlockSpec((_NT, H), lambda i, j: (j, 0)),
    ],
    out_specs=pl.BlockSpec((_MT, _NT), lambda i, j: (i, j)),
    out_shape=jax.ShapeDtypeStruct((M, CB), jnp.float32),
)


from jax import lax
from jax.experimental.pallas import tpu_sc as plsc

_NC = 2
_NS = 16
_NW = _NC * _NS
_SPW = B // _NW
_NCHUNK = CB // 16


def _sc_lsa_kernel(dist_hbm, cb_hbm, out_hbm, q_hbm, bufa_ref, bufb_ref, gbuf_ref, v_ref,
                   sh_ref, path_ref, rem_ref, r4c_ref, c4r_ref, u_s, sr_s,
                   qrows_ref, c4r_s, vnz_s, sema, semb, semq):
    wid = lax.axis_index("s") * _NC + lax.axis_index("c")
    lane = jax.lax.broadcasted_iota(jnp.int32, (16,), 0)
    inf = jnp.float32(jnp.inf)
    inf16 = jnp.full((16,), jnp.inf, jnp.float32)
    zero16i = jnp.zeros((16,), jnp.int32)

    def ds16(c):
        return pl.ds(c * 16, 16)

    def wr16(ref, pos, val):
        sl = pl.ds((pos // 16) * 16, 16)
        ref[sl] = jnp.where(lane == (pos % 16), val, ref[sl])

    def rd16i(ref, pos):
        vec = ref[pl.ds((pos // 16) * 16, 16)]
        return jnp.sum(jnp.where(lane == (pos % 16), vec, 0))

    def rd16f(ref, pos):
        vec = ref[pl.ds((pos // 16) * 16, 16)]
        return jnp.min(jnp.where(lane == (pos % 16), vec, inf))

    for so in range(_SPW):
        sample = wid * _SPW + so
        pltpu.async_copy(dist_hbm.at[sample * OBJ], bufa_ref, sema)

        def init_chunk(c, carry):
            v_ref[ds16(c)] = jnp.zeros((16,), jnp.float32)
            r4c_ref[ds16(c)] = jnp.full((16,), -1, jnp.int32)
            return carry

        lax.fori_loop(0, _NCHUNK, init_chunk, 0, unroll=8)

        def init_small(rr, carry):
            u_s[rr] = jnp.float32(0.0)
            c4r_s[rr] = jnp.int32(-1)
            return carry

        lax.fori_loop(0, OBJ, init_small, 0)
        vnz_s[0] = jnp.int32(0)
        c4r_ref[pl.ds(0, 16)] = jnp.full((16,), -1, jnp.int32)
        c4r_ref[pl.ds(16, 16)] = jnp.full((16,), -1, jnp.int32)

        def row_body(cur_row, crow_ref, sem, obuf_ref, osem):
            pltpu.make_async_copy(
                dist_hbm.at[sample * OBJ + cur_row], crow_ref, sem).wait()

            @pl.when(cur_row + 1 < OBJ)
            def _():
                pltpu.async_copy(
                    dist_hbm.at[sample * OBJ + cur_row + 1], obuf_ref, osem)

            def fp_chunk(c, st):
                rm, rc = st
                x = crow_ref[ds16(c)] - v_ref[ds16(c)]
                lt = x < rm
                rm = jnp.where(lt, x, rm)
                rc = jnp.where(lt, c, rc)
                return (rm, rc)

            def fp_chunk_vz(c, st):
                rm, rc = st
                x = crow_ref[ds16(c)]
                lt = x < rm
                rm = jnp.where(lt, x, rm)
                rc = jnp.where(lt, c, rc)
                return (rm, rc)

            rm, rc = lax.cond(
                vnz_s[0] == 0,
                lambda: lax.fori_loop(0, _NCHUNK, fp_chunk_vz,
                                      (inf16, zero16i), unroll=16),
                lambda: lax.fori_loop(0, _NCHUNK, fp_chunk,
                                      (inf16, zero16i), unroll=16))
            mv0 = jnp.min(rm)
            j0 = jnp.min(jnp.where(rm == mv0, rc * 16 + lane, _BIG))
            taken = rd16i(r4c_ref, j0) != -1

            @pl.when(jnp.logical_not(taken))
            def _():
                u_s[cur_row] = u_s[cur_row] + mv0
                c4r_s[cur_row] = j0
                wr16(c4r_ref, cur_row, j0)
                wr16(r4c_ref, j0, cur_row)

            @pl.when(taken)
            def _():
                def ginit(c, carryg):
                    sh_ref[ds16(c)] = crow_ref[ds16(c)] - v_ref[ds16(c)]
                    path_ref[ds16(c)] = jnp.full((16,), 0, jnp.int32) + cur_row
                    rem_ref[ds16(c)] = jnp.full((16,), 1, jnp.int32)
                    return carryg

                lax.fori_loop(0, _NCHUNK, ginit, 0, unroll=8)
                wr16(rem_ref, j0, jnp.int32(0))

                def clear_sr(rr, carryg):
                    sr_s[rr] = jnp.int32(0)
                    return carryg

                lax.fori_loop(0, OBJ, clear_sr, 0)
                sr_s[cur_row] = jnp.int32(1)

                def search_cond(s):
                    return s[1] == -1

                def search_body(s):
                    i, sink, min_val, nsr = s
                    sr_s[i] = jnp.int32(1)
                    pltpu.sync_copy(dist_hbm.at[sample * OBJ + i], gbuf_ref)
                    u_i = u_s[i]

                    def schunk(c, st):
                        srm, src_ = st
                        x = ((min_val + gbuf_ref[ds16(c)]) - u_i) - v_ref[ds16(c)]
                        sh = sh_ref[ds16(c)]
                        remc = rem_ref[ds16(c)]
                        remb = remc != 0
                        upd = remb & (x < sh)
                        sh2 = jnp.where(upd, x, sh)
                        sh_ref[ds16(c)] = sh2
                        path_ref[ds16(c)] = jnp.where(upd, i, path_ref[ds16(c)])
                        masked = jnp.where(remb, sh2, inf)
                        lt = masked < srm
                        srm = jnp.where(lt, masked, srm)
                        src_ = jnp.where(lt, c, src_)
                        return (srm, src_)

                    srm, src_ = lax.fori_loop(0, _NCHUNK, schunk,
                                              (inf16, zero16i), unroll=8)
                    mv = jnp.min(srm)
                    j = jnp.min(jnp.where(srm == mv, src_ * 16 + lane, _BIG))
                    wr16(rem_ref, j, jnp.int32(0))
                    r4c_j = rd16i(r4c_ref, j)
                    hit = r4c_j == -1
                    sink = jnp.where(hit, j, sink)
                    i = jnp.where(hit, i, r4c_j)
                    return (i, sink, mv, nsr + 1)

                init = (rd16i(r4c_ref, j0), jnp.int32(-1),
                        mv0, jnp.int32(1))
                _, sink, min_val, nsr = lax.while_loop(
                    search_cond, search_body, init)

                u_s[cur_row] = u_s[cur_row] + min_val

                @pl.when(nsr > 1)
                def _():
                    vnz_s[0] = jnp.int32(1)

                @pl.when(nsr > 1)
                def _():
                    def ex_body(rr, carryg):
                        take = (sr_s[rr] > 0) & (rr != cur_row)

                        @pl.when(take)
                        def _():
                            s_val = rd16f(sh_ref, c4r_s[rr])
                            u_s[rr] = u_s[rr] + (min_val - s_val)

                        return carryg

                    lax.fori_loop(0, OBJ, ex_body, 0)

                def vchunk(c, carryg):
                    t = min_val - sh_ref[ds16(c)]
                    remc = rem_ref[ds16(c)]
                    v_ref[ds16(c)] = jnp.where(remc == 0,
                                               v_ref[ds16(c)] - t,
                                               v_ref[ds16(c)])
                    return carryg

                lax.fori_loop(0, _NCHUNK, vchunk, 0, unroll=4)

                def aug_cond(s):
                    return jnp.logical_not(s[1])

                def aug_body(s):
                    j, done = s
                    i = rd16i(path_ref, j)
                    wr16(r4c_ref, j, i)
                    nj = c4r_s[i]
                    c4r_s[i] = j
                    wr16(c4r_ref, i, j)
                    done = i == cur_row
                    return (nj, done)

                lax.while_loop(aug_cond, aug_body, (sink, jnp.bool_(False)))

        def outer(kk, carry):
            row_body(kk * 2, bufa_ref, sema, bufb_ref, semb)
            row_body(kk * 2 + 1, bufb_ref, semb, bufa_ref, sema)
            return carry

        lax.fori_loop(0, OBJ // 2, outer, 0)
        pltpu.sync_copy(c4r_ref, out_hbm.at[sample])
        pltpu.async_copy(cb_hbm.at[c4r_ref], qrows_ref, semq).wait()
        pltpu.sync_copy(qrows_ref, q_hbm.at[sample])


_lsa_call_sc = functools.partial(
    pl.kernel,
    mesh=plsc.VectorSubcoreMesh(core_axis_name="c", subcore_axis_name="s"),
    compiler_params=pltpu.CompilerParams(needs_layout_passes=False),
    out_type=[jax.ShapeDtypeStruct((B, OBJ), jnp.int32),
              jax.ShapeDtypeStruct((B, OBJ, H), jnp.float32)],
    scratch_types=[
        pltpu.VMEM((CB,), jnp.float32),
        pltpu.VMEM((CB,), jnp.float32),
        pltpu.VMEM((CB,), jnp.float32),
        pltpu.VMEM((CB,), jnp.float32),
        pltpu.VMEM((CB,), jnp.float32),
        pltpu.VMEM((CB,), jnp.int32),
        pltpu.VMEM((CB,), jnp.int32),
        pltpu.VMEM((CB,), jnp.int32),
        pltpu.VMEM((OBJ,), jnp.int32),
        pltpu.SMEM((OBJ,), jnp.float32),
        pltpu.SMEM((OBJ,), jnp.int32),
        pltpu.VMEM((OBJ, H), jnp.float32),
        pltpu.SMEM((OBJ,), jnp.int32),
        pltpu.SMEM((1,), jnp.int32),
        pltpu.SemaphoreType.DMA,
        pltpu.SemaphoreType.DMA,
        pltpu.SemaphoreType.DMA,
    ],
)(_sc_lsa_kernel)


def _dec_body(q_ref, w_ref, b_ref, o_ref):
    k = pl.program_id(0)
    nk = pl.num_programs(0)
    p = jax.lax.dot_general(q_ref[...], w_ref[...], (((1,), (1,)), ((), ())),
                            preferred_element_type=jnp.float32)

    @pl.when(k == 0)
    def _():
        o_ref[...] = p

    @pl.when(k > 0)
    def _():
        o_ref[...] += p

    @pl.when(k == nk - 1)
    def _():
        o_ref[...] += b_ref[...][None, :]


_NK_D = 8
_BLK_D = (OBJ * H) // _NK_D

_dec = pl.pallas_call(
    _dec_body,
    grid=(_NK_D,),
    in_specs=[
        pl.BlockSpec((B, _BLK_D), lambda k: (0, k)),
        pl.BlockSpec((H, _BLK_D), lambda k: (0, k)),
        pl.BlockSpec((H,), lambda k: (0,)),
    ],
    out_specs=pl.BlockSpec((B, H), lambda k: (0, 0)),
    out_shape=jax.ShapeDtypeStruct((B, H), jnp.float32),
)


def kernel(x, codebook_w, enc_W, enc_b, dec_W, dec_b):
    e2d = _enc(x, enc_W, enc_b)
    ef = e2d.reshape(M, H)
    p = _prod(ef, codebook_w)
    se = jnp.sum(ef ** 2, axis=1, keepdims=True)
    sc = jnp.sum(codebook_w ** 2, axis=1)
    dist = jnp.sqrt(se + sc - 2.0 * p)
    _, q = _lsa_call_sc(dist, codebook_w)
    out = _dec(q.reshape(B, OBJ * H), dec_W, dec_b)
    e = e2d.reshape(B, OBJ, H)
    return (out, q, e)
